# baseline (speedup 1.0000x reference)
import jax
import jax.numpy as jnp
from jax.experimental import pallas as pl
from jax.experimental.pallas import tpu as pltpu


def _band_avg_kernel(m_ref, x_ref, o_ref):
    m = m_ref[...]
    scale = 1.0 / jnp.sum(m, axis=1, keepdims=True)
    sm = m * scale
    bt = x_ref.shape[0]
    for i in range(bt):
        o_ref[i] = jnp.dot(sm, x_ref[i], preferred_element_type=jnp.float32)


def kernel(x, freq_masks):
    b, t, f, c1, c2 = x.shape
    g = freq_masks.shape[0]
    bt = b * t
    c = c1 * c2
    xr = x.reshape(bt, f, c)

    BT = 25
    assert bt % BT == 0
    grid = (bt // BT,)

    out = pl.pallas_call(
        _band_avg_kernel,
        out_shape=jax.ShapeDtypeStruct((bt, g, c), jnp.float32),
        grid=grid,
        in_specs=[
            pl.BlockSpec((g, f), lambda i: (0, 0)),
            pl.BlockSpec((BT, f, c), lambda i: (i, 0, 0)),
        ],
        out_specs=pl.BlockSpec((BT, g, c), lambda i: (i, 0, 0)),
        compiler_params=pltpu.CompilerParams(
            dimension_semantics=("parallel",),
            vmem_limit_bytes=56 * 1024 * 1024,
        ),
        name="freq_band_avg",
    )(freq_masks, xr)

    return out.reshape(b, t, g, c1, c2)
